# baseline (speedup 1.0000x reference)
import functools

import jax
import jax.numpy as jnp
import numpy as np
from jax import lax
from jax.experimental import pallas as pl
from jax.experimental.pallas import tpu as pltpu
from jax.experimental.pallas import tpu_sc as plsc

_NUM_TRAIN_TIMESTEPS = 1000
_TABLE_PAD = 1024
_LANES = 16
_ROW = 128


def _coef_table_np():
    betas = np.linspace(np.float32(1e-4), np.float32(0.02),
                        _NUM_TRAIN_TIMESTEPS, dtype=np.float32)
    alphas_cumprod = np.cumprod((np.float32(1.0) - betas).astype(np.float32),
                                dtype=np.float32)
    sa = np.sqrt(alphas_cumprod).astype(np.float32)
    so = np.sqrt((np.float32(1.0) - alphas_cumprod).astype(np.float32))
    table = np.zeros((_TABLE_PAD, _ROW), np.float32)
    table[:_NUM_TRAIN_TIMESTEPS, 0] = sa
    table[:_NUM_TRAIN_TIMESTEPS, 1] = so
    return table


_TABLE_CONST = _coef_table_np()


def _sc_gather(table, timesteps):
    B = timesteps.shape[0]
    info = plsc.get_sparse_core_info()
    nc, ns = info.num_cores, info.num_subcores
    nw = nc * ns
    b_per_w = B // nw
    mesh = plsc.VectorSubcoreMesh(core_axis_name="c", subcore_axis_name="s")

    @functools.partial(
        pl.kernel,
        out_type=jax.ShapeDtypeStruct((B, _ROW), jnp.float32),
        mesh=mesh,
        scratch_types=[
            pltpu.VMEM((b_per_w,), jnp.int32),
            pltpu.VMEM((b_per_w, _ROW), jnp.float32),
            pltpu.SemaphoreType.DMA,
        ],
    )
    def gather_kernel(table_hbm, ts_hbm, out_hbm, idx_v, rows_v, sem):
        wid = lax.axis_index("s") * nc + lax.axis_index("c")
        base = wid * b_per_w
        pltpu.sync_copy(ts_hbm.at[pl.ds(base, b_per_w)], idx_v)
        pltpu.async_copy(table_hbm.at[idx_v], rows_v, sem).wait()
        pltpu.sync_copy(rows_v, out_hbm.at[pl.ds(base, b_per_w)])

    return gather_kernel(table, timesteps)


def _tc_blend(coef2, xt, nt, block_r):
    D, B = xt.shape
    grid = (D // block_r,)

    def body(coef_ref, x_ref, n_ref, o_ref):
        sa = coef_ref[0:1, :]
        so = coef_ref[1:2, :]
        o_ref[...] = sa * x_ref[...] + so * n_ref[...]

    return pl.pallas_call(
        body,
        grid=grid,
        in_specs=[
            pl.BlockSpec((2, B), lambda i: (0, 0)),
            pl.BlockSpec((block_r, B), lambda i: (i, 0)),
            pl.BlockSpec((block_r, B), lambda i: (i, 0)),
        ],
        out_specs=pl.BlockSpec((block_r, B), lambda i: (i, 0)),
        out_shape=jax.ShapeDtypeStruct((D, B), jnp.float32),
    )(coef2, xt, nt)


def kernel(original_samples, noise, timesteps):
    B, C, H, W = original_samples.shape
    D = C * H * W
    table = jnp.asarray(_TABLE_CONST)
    coef = _sc_gather(table, timesteps.astype(jnp.int32))
    coef2 = coef[:, :2].T
    xt = original_samples.transpose(1, 2, 3, 0).reshape(D, B)
    nt = noise.transpose(1, 2, 3, 0).reshape(D, B)
    out = _tc_blend(coef2, xt, nt, block_r=2048)
    return out.reshape(C, H, W, B).transpose(3, 0, 1, 2)


# speedup vs baseline: 1.0006x; 1.0006x over previous
import functools

import jax
import jax.numpy as jnp
import numpy as np
from jax import lax
from jax.experimental import pallas as pl
from jax.experimental.pallas import tpu as pltpu
from jax.experimental.pallas import tpu_sc as plsc

_NUM_TRAIN_TIMESTEPS = 1000
_TABLE_PAD = 1024
_LANES = 16
_ROW = 128


def _coef_table_np():
    betas = np.linspace(np.float32(1e-4), np.float32(0.02),
                        _NUM_TRAIN_TIMESTEPS, dtype=np.float32)
    alphas_cumprod = np.cumprod((np.float32(1.0) - betas).astype(np.float32),
                                dtype=np.float32)
    sa = np.sqrt(alphas_cumprod).astype(np.float32)
    so = np.sqrt((np.float32(1.0) - alphas_cumprod).astype(np.float32))
    table = np.zeros((_TABLE_PAD, _ROW), np.float32)
    table[:_NUM_TRAIN_TIMESTEPS, 0] = sa
    table[:_NUM_TRAIN_TIMESTEPS, 1] = so
    return table


_TABLE_CONST = _coef_table_np()


def _sc_gather(table, timesteps):
    B = timesteps.shape[0]
    info = plsc.get_sparse_core_info()
    nc, ns = info.num_cores, info.num_subcores
    nw = nc * ns
    b_per_w = B // nw
    mesh = plsc.VectorSubcoreMesh(core_axis_name="c", subcore_axis_name="s")

    @functools.partial(
        pl.kernel,
        out_type=jax.ShapeDtypeStruct((B, _ROW), jnp.float32),
        mesh=mesh,
        scratch_types=[
            pltpu.VMEM((b_per_w,), jnp.int32),
            pltpu.VMEM((b_per_w, _ROW), jnp.float32),
            pltpu.SemaphoreType.DMA,
        ],
    )
    def gather_kernel(table_hbm, ts_hbm, out_hbm, idx_v, rows_v, sem):
        wid = lax.axis_index("s") * nc + lax.axis_index("c")
        base = wid * b_per_w
        pltpu.sync_copy(ts_hbm.at[pl.ds(base, b_per_w)], idx_v)
        pltpu.async_copy(table_hbm.at[idx_v], rows_v, sem).wait()
        pltpu.sync_copy(rows_v, out_hbm.at[pl.ds(base, b_per_w)])

    return gather_kernel(table, timesteps)


def _tc_blend(coef2, xt, nt, block_r):
    D, B = xt.shape
    grid = (D // block_r,)

    def body(coef_ref, x_ref, n_ref, o_ref):
        sa = coef_ref[0:1, :]
        so = coef_ref[1:2, :]
        o_ref[...] = sa * x_ref[...] + so * n_ref[...]

    return pl.pallas_call(
        body,
        grid=grid,
        in_specs=[
            pl.BlockSpec((2, B), lambda i: (0, 0)),
            pl.BlockSpec((block_r, B), lambda i: (i, 0)),
            pl.BlockSpec((block_r, B), lambda i: (i, 0)),
        ],
        out_specs=pl.BlockSpec((block_r, B), lambda i: (i, 0)),
        out_shape=jax.ShapeDtypeStruct((D, B), jnp.float32),
    )(coef2, xt, nt)


def kernel(original_samples, noise, timesteps):
    B, C, H, W = original_samples.shape
    D = C * H * W
    table = jnp.asarray(_TABLE_CONST)
    coef = _sc_gather(table, timesteps.astype(jnp.int32))
    coef2 = coef[:, :2].T
    xt = original_samples.transpose(1, 2, 3, 0).reshape(D, B)
    nt = noise.transpose(1, 2, 3, 0).reshape(D, B)
    out = _tc_blend(coef2, xt, nt, block_r=512)
    return out.reshape(C, H, W, B).transpose(3, 0, 1, 2)


# speedup vs baseline: 1.0279x; 1.0273x over previous
import functools

import jax
import jax.numpy as jnp
import numpy as np
from jax import lax
from jax.experimental import pallas as pl
from jax.experimental.pallas import tpu as pltpu
from jax.experimental.pallas import tpu_sc as plsc

_NUM_TRAIN_TIMESTEPS = 1000
_TABLE_PAD = 1024
_LANES = 16
_ROW = 128


def _coef_table_np():
    betas = np.linspace(np.float32(1e-4), np.float32(0.02),
                        _NUM_TRAIN_TIMESTEPS, dtype=np.float32)
    alphas_cumprod = np.cumprod((np.float32(1.0) - betas).astype(np.float32),
                                dtype=np.float32)
    sa = np.sqrt(alphas_cumprod).astype(np.float32)
    so = np.sqrt((np.float32(1.0) - alphas_cumprod).astype(np.float32))
    table = np.zeros((_TABLE_PAD, _ROW), np.float32)
    table[:_NUM_TRAIN_TIMESTEPS, 0] = sa
    table[:_NUM_TRAIN_TIMESTEPS, 1] = so
    return table


_TABLE_CONST = _coef_table_np()


def _sc_gather(table, timesteps):
    B = timesteps.shape[0]
    info = plsc.get_sparse_core_info()
    nc, ns = info.num_cores, info.num_subcores
    nw = nc * ns
    b_per_w = B // nw
    mesh = plsc.VectorSubcoreMesh(core_axis_name="c", subcore_axis_name="s")

    @functools.partial(
        pl.kernel,
        out_type=jax.ShapeDtypeStruct((B, _ROW), jnp.float32),
        mesh=mesh,
        scratch_types=[
            pltpu.VMEM((b_per_w,), jnp.int32),
            pltpu.VMEM((b_per_w, _ROW), jnp.float32),
            pltpu.SemaphoreType.DMA,
        ],
    )
    def gather_kernel(table_hbm, ts_hbm, out_hbm, idx_v, rows_v, sem):
        wid = lax.axis_index("s") * nc + lax.axis_index("c")
        base = wid * b_per_w
        pltpu.sync_copy(ts_hbm.at[pl.ds(base, b_per_w)], idx_v)
        pltpu.async_copy(table_hbm.at[idx_v], rows_v, sem).wait()
        pltpu.sync_copy(rows_v, out_hbm.at[pl.ds(base, b_per_w)])

    return gather_kernel(table, timesteps)


def _tc_blend(coef, xt, nt, block_r):
    D, B = xt.shape
    grid = (D // block_r,)

    def body(coef_ref, x_ref, n_ref, o_ref, coef_t_ref):
        @pl.when(pl.program_id(0) == 0)
        def _():
            coef_t_ref[...] = coef_ref[:, 0:8].T

        sa = coef_t_ref[0:1, :]
        so = coef_t_ref[1:2, :]
        o_ref[...] = sa * x_ref[...] + so * n_ref[...]

    return pl.pallas_call(
        body,
        grid=grid,
        in_specs=[
            pl.BlockSpec((B, _ROW), lambda i: (0, 0)),
            pl.BlockSpec((block_r, B), lambda i: (i, 0)),
            pl.BlockSpec((block_r, B), lambda i: (i, 0)),
        ],
        out_specs=pl.BlockSpec((block_r, B), lambda i: (i, 0)),
        out_shape=jax.ShapeDtypeStruct((D, B), jnp.float32),
        scratch_shapes=[pltpu.VMEM((8, B), jnp.float32)],
    )(coef, xt, nt)


def kernel(original_samples, noise, timesteps):
    B, C, H, W = original_samples.shape
    D = C * H * W
    table = jnp.asarray(_TABLE_CONST)
    coef = _sc_gather(table, timesteps.astype(jnp.int32))
    xt = original_samples.transpose(1, 2, 3, 0).reshape(D, B)
    nt = noise.transpose(1, 2, 3, 0).reshape(D, B)
    out = _tc_blend(coef, xt, nt, block_r=1024)
    return out.reshape(C, H, W, B).transpose(3, 0, 1, 2)


# speedup vs baseline: 1.2368x; 1.2033x over previous
import functools

import jax
import jax.numpy as jnp
import numpy as np
from jax import lax
from jax.experimental import pallas as pl
from jax.experimental.pallas import tpu as pltpu
from jax.experimental.pallas import tpu_sc as plsc

_NUM_TRAIN_TIMESTEPS = 1000
_TABLE_PAD = 1024
_LANES = 16
_ROW = 128


def _coef_table_np():
    betas = np.linspace(np.float32(1e-4), np.float32(0.02),
                        _NUM_TRAIN_TIMESTEPS, dtype=np.float32)
    alphas_cumprod = np.cumprod((np.float32(1.0) - betas).astype(np.float32),
                                dtype=np.float32)
    sa = np.sqrt(alphas_cumprod).astype(np.float32)
    so = np.sqrt((np.float32(1.0) - alphas_cumprod).astype(np.float32))
    table = np.zeros((_TABLE_PAD, _ROW), np.float32)
    table[:_NUM_TRAIN_TIMESTEPS, 0] = sa
    table[:_NUM_TRAIN_TIMESTEPS, 1] = so
    return table


_TABLE_CONST = _coef_table_np()


def _sc_gather(table, timesteps):
    B = timesteps.shape[0]
    info = plsc.get_sparse_core_info()
    nc, ns = info.num_cores, info.num_subcores
    nw = nc * ns
    b_per_w = B // nw
    mesh = plsc.VectorSubcoreMesh(core_axis_name="c", subcore_axis_name="s")

    @functools.partial(
        pl.kernel,
        out_type=jax.ShapeDtypeStruct((B, _ROW), jnp.float32),
        mesh=mesh,
        scratch_types=[
            pltpu.VMEM((b_per_w,), jnp.int32),
            pltpu.VMEM((b_per_w, _ROW), jnp.float32),
            pltpu.SemaphoreType.DMA,
        ],
    )
    def gather_kernel(table_hbm, ts_hbm, out_hbm, idx_v, rows_v, sem):
        wid = lax.axis_index("s") * nc + lax.axis_index("c")
        base = wid * b_per_w
        pltpu.sync_copy(ts_hbm.at[pl.ds(base, b_per_w)], idx_v)
        pltpu.async_copy(table_hbm.at[idx_v], rows_v, sem).wait()
        pltpu.sync_copy(rows_v, out_hbm.at[pl.ds(base, b_per_w)])

    return gather_kernel(table, timesteps)


def _tc_blend(coef, xt, nt, block_r):
    D, B = xt.shape
    grid = (D // block_r,)

    def body(coef_ref, x_ref, n_ref, o_ref, coef_t_ref):
        @pl.when(pl.program_id(0) == 0)
        def _():
            coef_t_ref[...] = coef_ref[:, 0:8].T

        sa = coef_t_ref[0:1, :]
        so = coef_t_ref[1:2, :]
        o_ref[...] = sa * x_ref[...] + so * n_ref[...]

    return pl.pallas_call(
        body,
        grid=grid,
        in_specs=[
            pl.BlockSpec((B, _ROW), lambda i: (0, 0)),
            pl.BlockSpec((block_r, B), lambda i: (i, 0)),
            pl.BlockSpec((block_r, B), lambda i: (i, 0)),
        ],
        out_specs=pl.BlockSpec((block_r, B), lambda i: (i, 0)),
        out_shape=jax.ShapeDtypeStruct((D, B), jnp.float32),
        scratch_shapes=[pltpu.VMEM((8, B), jnp.float32)],
    )(coef, xt, nt)


def kernel(original_samples, noise, timesteps):
    B, C, H, W = original_samples.shape
    D = C * H * W
    table = jnp.asarray(_TABLE_CONST)
    coef = jnp.take(table, timesteps.astype(jnp.int32), axis=0)
    xt = original_samples.transpose(1, 2, 3, 0).reshape(D, B)
    nt = noise.transpose(1, 2, 3, 0).reshape(D, B)
    out = _tc_blend(coef, xt, nt, block_r=1024)
    return out.reshape(C, H, W, B).transpose(3, 0, 1, 2)
